# baseline (speedup 1.0000x reference)
import math

import jax
import jax.numpy as jnp
from jax.experimental import pallas as pl

_B = 64
_K = 100000
_ROWS = 8
_LGAMMA_K = math.lgamma(float(_K))
_LN2 = math.log(2.0)
_LOG2E = 1.0 / _LN2
_NEG_LOG2_LN2 = -math.log2(_LN2)


def _body(temp_ref, logits_ref, u_ref, sample_ref, lp_ref):
    temp = temp_ref[0, 0]
    it = 1.0 / temp
    n2 = -jnp.log2(u_ref[...])
    g2 = _NEG_LOG2_LN2 - jnp.log2(n2)
    e = jnp.exp2(logits_ref[...] * (_LOG2E * it) + g2 * it)
    s = jnp.sum(e, axis=-1, keepdims=True)
    sample_ref[...] = e * (1.0 / s)
    sum_g2 = jnp.sum(g2, axis=-1, keepdims=True)
    sum_n2 = jnp.sum(n2, axis=-1, keepdims=True)
    log_scale = _LGAMMA_K + (_K - 1.0) * jnp.log(temp)
    lp_ref[...] = (-_LN2 * sum_g2
                   - _K * (jnp.log(sum_n2) + math.log(_LN2))
                   + log_scale)


def kernel(logits, gumbel_u, temperature):
    temp2d = temperature.reshape(1, 1)
    grid = (_B // _ROWS,)
    sample, lp = pl.pallas_call(
        _body,
        grid=grid,
        in_specs=[
            pl.BlockSpec((1, 1), lambda i: (0, 0)),
            pl.BlockSpec((_ROWS, _K), lambda i: (i, 0)),
            pl.BlockSpec((_ROWS, _K), lambda i: (i, 0)),
        ],
        out_specs=[
            pl.BlockSpec((_ROWS, _K), lambda i: (i, 0)),
            pl.BlockSpec((_ROWS, 1), lambda i: (i, 0)),
        ],
        out_shape=[
            jax.ShapeDtypeStruct((_B, _K), jnp.float32),
            jax.ShapeDtypeStruct((_B, 1), jnp.float32),
        ],
    )(temp2d, logits, gumbel_u)
    return sample, lp.reshape(_B)
